# baseline (speedup 1.0000x reference)
import functools

import jax
import jax.numpy as jnp
from jax import lax
from jax.experimental import pallas as pl
from jax.experimental.pallas import tpu as pltpu
from jax.experimental.pallas import tpu_sc as plsc

L = 16
NC = 2
NS = 16
NW = NC * NS
B = 16384
K = 16
BPW = B // NW
NBLK = BPW // L
NROW = 1000000
MVN = 131072
GRID = (NROW + MVN - 1) // MVN
NROWP = GRID * MVN
CSPLIT = 917504
CPW = CSPLIT // NW
NCH = 14
TPC = 16
CHW = TPC * 128


def _mv_body(wt_ref, ht2_ref, wu_ref, wv_ref, b_ref, a_hbm, ct_hbm,
             abuf, tbuf, sem_ab, sem_t):
    i = pl.program_id(0)
    slot = lax.rem(i, 2)

    @pl.when(i >= 2)
    def _():
        pltpu.make_async_copy(abuf.at[slot], a_hbm.at[pl.ds(0, MVN)],
                              sem_ab.at[slot]).wait()

    abuf[slot, :] = jnp.sum(wt_ref[...] * wu_ref[...], axis=0) + b_ref[0, 0]
    pltpu.async_copy(abuf.at[slot], a_hbm.at[pl.ds(i * MVN, MVN)],
                     sem_ab.at[slot])

    @pl.when(i == 0)
    def _():
        tbuf[...] = jnp.sum(ht2_ref[...] * wv_ref[...], axis=0)
        pltpu.async_copy(tbuf, ct_hbm.at[pl.ds(0, MVN)], sem_t)

    @pl.when(i == GRID - 1)
    def _():
        for s in range(2):
            pltpu.make_async_copy(abuf.at[s], a_hbm.at[pl.ds(0, MVN)],
                                  sem_ab.at[s]).wait()
        pltpu.make_async_copy(tbuf, ct_hbm.at[pl.ds(0, MVN)], sem_t).wait()


_mv_call = pl.pallas_call(
    _mv_body,
    grid=(GRID,),
    in_specs=[
        pl.BlockSpec((K, MVN), lambda i: (0, i)),
        pl.BlockSpec((K, MVN), lambda i: (0, GRID - 1)),
        pl.BlockSpec((K, 1), lambda i: (0, 0)),
        pl.BlockSpec((K, 1), lambda i: (0, 0)),
        pl.BlockSpec((1, 1), lambda i: (0, 0)),
    ],
    out_specs=[
        pl.BlockSpec(memory_space=pl.ANY),
        pl.BlockSpec(memory_space=pl.ANY),
    ],
    out_shape=[
        jax.ShapeDtypeStruct((NROWP,), jnp.float32),
        jax.ShapeDtypeStruct((MVN,), jnp.float32),
    ],
    scratch_shapes=[
        pltpu.VMEM((2, MVN), jnp.float32),
        pltpu.VMEM((MVN,), jnp.float32),
        pltpu.SemaphoreType.DMA((2,)),
        pltpu.SemaphoreType.DMA,
    ],
)


def _cmv_body(ht_hbm, wvb_hbm, c_hbm, b00, b01, b10, b11, wvb_v, out_v,
              sem0, sem1):
    wid = lax.axis_index("s") * NC + lax.axis_index("c")
    base = wid * CPW
    pltpu.sync_copy(wvb_hbm, wvb_v)
    bufs = [(b00, b01), (b10, b11)]
    sems = [sem0, sem1]

    def issue(ch, par):
        col0 = base + ch * CHW
        lo, hi = bufs[par]

        def body(t, _):
            pltpu.async_copy(
                ht_hbm.at[pl.ds(0, 8), pl.ds(col0 + t * 128, 128)],
                lo.at[t], sems[par])
            pltpu.async_copy(
                ht_hbm.at[pl.ds(8, 8), pl.ds(col0 + t * 128, 128)],
                hi.at[t], sems[par])
            return _
        lax.fori_loop(0, TPC, body, None)

    def drain(par):
        lo, hi = bufs[par]

        def body(t, _):
            pltpu.make_async_copy(
                ht_hbm.at[pl.ds(0, 8), pl.ds(0, 128)],
                lo.at[t], sems[par]).wait()
            pltpu.make_async_copy(
                ht_hbm.at[pl.ds(0, 8), pl.ds(0, 128)],
                hi.at[t], sems[par]).wait()
            return _
        lax.fori_loop(0, TPC, body, None)

    def compute(ch, par):
        off0 = ch * CHW
        lo, hi = bufs[par]

        def tile(t, _):
            for j in range(8):
                acc = jnp.zeros((L,), jnp.float32)
                for r in range(8):
                    acc = acc + (lo[t, r, pl.ds(j * 16, 16)]
                                 * wvb_v[r, pl.ds(0, 16)])
                for r in range(8):
                    acc = acc + (hi[t, r, pl.ds(j * 16, 16)]
                                 * wvb_v[8 + r, pl.ds(0, 16)])
                out_v[pl.ds(off0 + t * 128 + j * 16, 16)] = acc
            return _
        lax.fori_loop(0, TPC, tile, None)

    issue(0, 0)

    def pair(cp, _):
        ch0 = cp * 2
        drain(0)
        issue(ch0 + 1, 1)
        compute(ch0, 0)
        drain(1)

        @pl.when(ch0 + 2 < NCH)
        def _():
            issue(ch0 + 2, 0)
        compute(ch0 + 1, 1)
        return _

    lax.fori_loop(0, NCH // 2, pair, None)
    pltpu.sync_copy(out_v, c_hbm.at[pl.ds(base, CPW)])


_sc_cmv = functools.partial(
    pl.kernel,
    out_type=jax.ShapeDtypeStruct((CSPLIT,), jnp.float32),
    mesh=plsc.VectorSubcoreMesh(core_axis_name="c", subcore_axis_name="s"),
    compiler_params=pltpu.CompilerParams(
        use_tc_tiling_on_sc=True, needs_layout_passes=False),
    scratch_types=[
        pltpu.VMEM((TPC, 8, 128), jnp.float32),
        pltpu.VMEM((TPC, 8, 128), jnp.float32),
        pltpu.VMEM((TPC, 8, 128), jnp.float32),
        pltpu.VMEM((TPC, 8, 128), jnp.float32),
        pltpu.VMEM((16, 128), jnp.float32),
        pltpu.VMEM((CPW,), jnp.float32),
        pltpu.SemaphoreType.DMA,
        pltpu.SemaphoreType.DMA,
    ],
)(_cmv_body)


def _sc_body(u_hbm, v_hbm, a_hbm, ch_hbm, ct_hbm, out_hbm,
             uidx_v, vidx_v, vh_v, vt_v, av, chv, ctv, out_v,
             sem_a, sem_h, sem_t):
    wid = lax.axis_index("s") * NC + lax.axis_index("c")
    base = wid * BPW

    pltpu.sync_copy(u_hbm.at[pl.ds(base, BPW)], uidx_v)
    pltpu.sync_copy(v_hbm.at[pl.ds(base, BPW)], vidx_v)

    def split(i, _):
        vv = vidx_v[pl.ds(i * L, L)]
        vh_v[pl.ds(i * L, L)] = jnp.minimum(vv, CSPLIT - 1)
        vt_v[pl.ds(i * L, L)] = jnp.maximum(vv - CSPLIT, 0)
        return _

    lax.fori_loop(0, NBLK, split, None)

    ca = pltpu.async_copy(a_hbm.at[uidx_v], av, sem_a)
    cb = pltpu.async_copy(ch_hbm.at[vh_v], chv, sem_h)
    cc = pltpu.async_copy(ct_hbm.at[vt_v], ctv, sem_t)
    ca.wait()
    cb.wait()
    cc.wait()

    def block(i, _):
        sl = pl.ds(i * L, L)
        cval = jnp.where(vidx_v[sl] < CSPLIT, chv[sl], ctv[sl])
        z = av[sl] + cval
        out_v[sl] = 1.0 / (1.0 + jnp.exp(-z))
        return _

    lax.fori_loop(0, NBLK, block, None)
    pltpu.sync_copy(out_v, out_hbm.at[pl.ds(base, BPW)])


_sc_lookup = functools.partial(
    pl.kernel,
    out_type=jax.ShapeDtypeStruct((B,), jnp.float32),
    mesh=plsc.VectorSubcoreMesh(core_axis_name="c", subcore_axis_name="s"),
    compiler_params=pltpu.CompilerParams(
        use_tc_tiling_on_sc=False, needs_layout_passes=False),
    scratch_types=[
        pltpu.VMEM((BPW,), jnp.int32),
        pltpu.VMEM((BPW,), jnp.int32),
        pltpu.VMEM((BPW,), jnp.int32),
        pltpu.VMEM((BPW,), jnp.int32),
        pltpu.VMEM((BPW,), jnp.float32),
        pltpu.VMEM((BPW,), jnp.float32),
        pltpu.VMEM((BPW,), jnp.float32),
        pltpu.VMEM((BPW,), jnp.float32),
        pltpu.SemaphoreType.DMA,
        pltpu.SemaphoreType.DMA,
        pltpu.SemaphoreType.DMA,
    ],
)(_sc_body)


def kernel(x, W, H, lin_w, lin_b):
    u_idx = x[:, 0]
    v_idx = x[:, 1]
    wu = lin_w[:, :K].reshape(K, 1)
    wv = lin_w[:, K:].reshape(K, 1)
    bias = lin_b.reshape(1, 1)
    wvb = jnp.broadcast_to(lin_w[0, K:].reshape(K, 1), (K, 128))
    ht = H.T
    c_head = _sc_cmv(ht, wvb)
    a, c_tail = _mv_call(W.T, ht, wu, wv, bias)
    return _sc_lookup(u_idx, v_idx, a, c_head, c_tail)


# speedup vs baseline: 1.8795x; 1.8795x over previous
import functools

import jax
import jax.numpy as jnp
from jax import lax
from jax.experimental import pallas as pl
from jax.experimental.pallas import tpu as pltpu
from jax.experimental.pallas import tpu_sc as plsc

L = 16
NC = 2
NS = 16
NW = NC * NS
B = 16384
K = 16
BPW = B // NW
NBLK = BPW // L
NROW = 1000000
MVN = 131072
GRID = (NROW + MVN - 1) // MVN
NROWP = GRID * MVN
CSPLIT = 917504
CPW = CSPLIT // NW
NCH = 14
TPC = 16
CHW = TPC * 128


def _mv_body(wt_ref, ht2_ref, wu_ref, wv_ref, b_ref, a_hbm, ct_hbm,
             abuf, tbuf, sem_ab, sem_t):
    i = pl.program_id(0)
    slot = lax.rem(i, 2)

    @pl.when(i >= 2)
    def _():
        pltpu.make_async_copy(abuf.at[slot], a_hbm.at[pl.ds(0, MVN)],
                              sem_ab.at[slot]).wait()

    abuf[slot, :] = jnp.sum(wt_ref[...] * wu_ref[...], axis=0) + b_ref[0, 0]
    pltpu.async_copy(abuf.at[slot], a_hbm.at[pl.ds(i * MVN, MVN)],
                     sem_ab.at[slot])

    @pl.when(i == 0)
    def _():
        tbuf[...] = jnp.sum(ht2_ref[...] * wv_ref[...], axis=0)
        pltpu.async_copy(tbuf, ct_hbm.at[pl.ds(0, MVN)], sem_t)

    @pl.when(i == GRID - 1)
    def _():
        for s in range(2):
            pltpu.make_async_copy(abuf.at[s], a_hbm.at[pl.ds(0, MVN)],
                                  sem_ab.at[s]).wait()
        pltpu.make_async_copy(tbuf, ct_hbm.at[pl.ds(0, MVN)], sem_t).wait()


_mv_call = pl.pallas_call(
    _mv_body,
    grid=(GRID,),
    in_specs=[
        pl.BlockSpec((K, MVN), lambda i: (0, i)),
        pl.BlockSpec((K, MVN), lambda i: (0, GRID - 1)),
        pl.BlockSpec((K, 1), lambda i: (0, 0)),
        pl.BlockSpec((K, 1), lambda i: (0, 0)),
        pl.BlockSpec((1, 1), lambda i: (0, 0)),
    ],
    out_specs=[
        pl.BlockSpec(memory_space=pl.ANY),
        pl.BlockSpec(memory_space=pl.ANY),
    ],
    out_shape=[
        jax.ShapeDtypeStruct((NROWP,), jnp.float32),
        jax.ShapeDtypeStruct((MVN,), jnp.float32),
    ],
    scratch_shapes=[
        pltpu.VMEM((2, MVN), jnp.float32),
        pltpu.VMEM((MVN,), jnp.float32),
        pltpu.SemaphoreType.DMA((2,)),
        pltpu.SemaphoreType.DMA,
    ],
)


def _cmv_body(ht_hbm, wvb_hbm, c_hbm, b00, b01, b10, b11, wvb_v, out_v,
              sem0, sem1):
    wid = lax.axis_index("s") * NC + lax.axis_index("c")
    base = wid * CPW
    pltpu.sync_copy(wvb_hbm, wvb_v)
    bufs = [(b00, b01), (b10, b11)]
    sems = [sem0, sem1]

    def issue(ch, par):
        col0 = base + ch * CHW
        lo, hi = bufs[par]
        pltpu.async_copy(ht_hbm.at[pl.ds(0, 8), pl.ds(col0, CHW)],
                         lo, sems[par])
        pltpu.async_copy(ht_hbm.at[pl.ds(8, 8), pl.ds(col0, CHW)],
                         hi, sems[par])

    def drain(par):
        lo, hi = bufs[par]
        pltpu.make_async_copy(ht_hbm.at[pl.ds(0, 8), pl.ds(0, CHW)],
                              lo, sems[par]).wait()
        pltpu.make_async_copy(ht_hbm.at[pl.ds(0, 8), pl.ds(0, CHW)],
                              hi, sems[par]).wait()

    def compute(ch, par):
        off0 = ch * CHW
        lo, hi = bufs[par]

        def tile(t, _):
            for j in range(8):
                acc = jnp.zeros((L,), jnp.float32)
                for r in range(8):
                    acc = acc + (lo[r, pl.ds(t * 128 + j * 16, 16)]
                                 * wvb_v[r, pl.ds(0, 16)])
                for r in range(8):
                    acc = acc + (hi[r, pl.ds(t * 128 + j * 16, 16)]
                                 * wvb_v[8 + r, pl.ds(0, 16)])
                out_v[pl.ds(off0 + t * 128 + j * 16, 16)] = acc
            return _
        lax.fori_loop(0, TPC, tile, None)

    issue(0, 0)

    def pair(cp, _):
        ch0 = cp * 2
        drain(0)
        issue(ch0 + 1, 1)
        compute(ch0, 0)
        drain(1)

        @pl.when(ch0 + 2 < NCH)
        def _():
            issue(ch0 + 2, 0)
        compute(ch0 + 1, 1)
        return _

    lax.fori_loop(0, NCH // 2, pair, None)
    pltpu.sync_copy(out_v, c_hbm.at[pl.ds(base, CPW)])


_sc_cmv = functools.partial(
    pl.kernel,
    out_type=jax.ShapeDtypeStruct((CSPLIT,), jnp.float32),
    mesh=plsc.VectorSubcoreMesh(core_axis_name="c", subcore_axis_name="s"),
    compiler_params=pltpu.CompilerParams(
        use_tc_tiling_on_sc=True, needs_layout_passes=False),
    scratch_types=[
        pltpu.VMEM((8, CHW), jnp.float32),
        pltpu.VMEM((8, CHW), jnp.float32),
        pltpu.VMEM((8, CHW), jnp.float32),
        pltpu.VMEM((8, CHW), jnp.float32),
        pltpu.VMEM((16, 128), jnp.float32),
        pltpu.VMEM((CPW,), jnp.float32),
        pltpu.SemaphoreType.DMA,
        pltpu.SemaphoreType.DMA,
    ],
)(_cmv_body)


def _sc_body(u_hbm, v_hbm, a_hbm, ch_hbm, ct_hbm, out_hbm,
             uidx_v, vidx_v, vh_v, vt_v, av, chv, ctv, out_v,
             sem_a, sem_h, sem_t):
    wid = lax.axis_index("s") * NC + lax.axis_index("c")
    base = wid * BPW

    pltpu.sync_copy(u_hbm.at[pl.ds(base, BPW)], uidx_v)
    pltpu.sync_copy(v_hbm.at[pl.ds(base, BPW)], vidx_v)

    def split(i, _):
        vv = vidx_v[pl.ds(i * L, L)]
        vh_v[pl.ds(i * L, L)] = lax.rem(vv, CSPLIT)
        vt_v[pl.ds(i * L, L)] = lax.bitwise_and(vv - CSPLIT, MVN - 1)
        return _

    lax.fori_loop(0, NBLK, split, None)

    ca = pltpu.async_copy(a_hbm.at[uidx_v], av, sem_a)
    cb = pltpu.async_copy(ch_hbm.at[vh_v], chv, sem_h)
    cc = pltpu.async_copy(ct_hbm.at[vt_v], ctv, sem_t)
    ca.wait()
    cb.wait()
    cc.wait()

    def block(i, _):
        sl = pl.ds(i * L, L)
        cval = jnp.where(vidx_v[sl] < CSPLIT, chv[sl], ctv[sl])
        z = av[sl] + cval
        out_v[sl] = 1.0 / (1.0 + jnp.exp(-z))
        return _

    lax.fori_loop(0, NBLK, block, None)
    pltpu.sync_copy(out_v, out_hbm.at[pl.ds(base, BPW)])


_sc_lookup = functools.partial(
    pl.kernel,
    out_type=jax.ShapeDtypeStruct((B,), jnp.float32),
    mesh=plsc.VectorSubcoreMesh(core_axis_name="c", subcore_axis_name="s"),
    compiler_params=pltpu.CompilerParams(
        use_tc_tiling_on_sc=False, needs_layout_passes=False),
    scratch_types=[
        pltpu.VMEM((BPW,), jnp.int32),
        pltpu.VMEM((BPW,), jnp.int32),
        pltpu.VMEM((BPW,), jnp.int32),
        pltpu.VMEM((BPW,), jnp.int32),
        pltpu.VMEM((BPW,), jnp.float32),
        pltpu.VMEM((BPW,), jnp.float32),
        pltpu.VMEM((BPW,), jnp.float32),
        pltpu.VMEM((BPW,), jnp.float32),
        pltpu.SemaphoreType.DMA,
        pltpu.SemaphoreType.DMA,
        pltpu.SemaphoreType.DMA,
    ],
)(_sc_body)


def kernel(x, W, H, lin_w, lin_b):
    u_idx = x[:, 0]
    v_idx = x[:, 1]
    wu = lin_w[:, :K].reshape(K, 1)
    wv = lin_w[:, K:].reshape(K, 1)
    bias = lin_b.reshape(1, 1)
    wvb = jnp.broadcast_to(lin_w[0, K:].reshape(K, 1), (K, 128))
    ht = H.T
    c_head = _sc_cmv(ht, wvb)
    a, c_tail = _mv_call(W.T, ht, wu, wv, bias)
    return _sc_lookup(u_idx, v_idx, a, c_head, c_tail)


# speedup vs baseline: 2.1620x; 1.1503x over previous
import functools

import jax
import jax.numpy as jnp
from jax import lax
from jax.experimental import pallas as pl
from jax.experimental.pallas import tpu as pltpu
from jax.experimental.pallas import tpu_sc as plsc

L = 16
NC = 2
NS = 16
NW = NC * NS
B = 16384
K = 16
BPW = B // NW
NBLK = BPW // L
NROW = 1000000
MVN = 131072
GRID = (NROW + MVN - 1) // MVN
NROWP = GRID * MVN


def _mv_body(wt_ref, ht_ref, wu_ref, wv_ref, b_ref, a_hbm, c_hbm,
             abuf, cbuf, sem_ab, sem_cb):
    i = pl.program_id(0)
    slot = lax.rem(i, 2)

    @pl.when(i >= 2)
    def _():
        pltpu.make_async_copy(abuf.at[slot], a_hbm.at[pl.ds(0, MVN)],
                              sem_ab.at[slot]).wait()
        pltpu.make_async_copy(cbuf.at[slot], c_hbm.at[pl.ds(0, MVN)],
                              sem_cb.at[slot]).wait()

    abuf[slot, :] = jnp.sum(wt_ref[...] * wu_ref[...], axis=0) + b_ref[0, 0]
    cbuf[slot, :] = jnp.sum(ht_ref[...] * wv_ref[...], axis=0)
    pltpu.async_copy(abuf.at[slot], a_hbm.at[pl.ds(i * MVN, MVN)],
                     sem_ab.at[slot])
    pltpu.async_copy(cbuf.at[slot], c_hbm.at[pl.ds(i * MVN, MVN)],
                     sem_cb.at[slot])

    @pl.when(i == GRID - 1)
    def _():
        for s in range(2):
            pltpu.make_async_copy(abuf.at[s], a_hbm.at[pl.ds(0, MVN)],
                                  sem_ab.at[s]).wait()
            pltpu.make_async_copy(cbuf.at[s], c_hbm.at[pl.ds(0, MVN)],
                                  sem_cb.at[s]).wait()


_mv_call = pl.pallas_call(
    _mv_body,
    grid=(GRID,),
    in_specs=[
        pl.BlockSpec((K, MVN), lambda i: (0, i)),
        pl.BlockSpec((K, MVN), lambda i: (0, i)),
        pl.BlockSpec((K, 1), lambda i: (0, 0)),
        pl.BlockSpec((K, 1), lambda i: (0, 0)),
        pl.BlockSpec((1, 1), lambda i: (0, 0)),
    ],
    out_specs=[
        pl.BlockSpec(memory_space=pl.ANY),
        pl.BlockSpec(memory_space=pl.ANY),
    ],
    out_shape=[
        jax.ShapeDtypeStruct((NROWP,), jnp.float32),
        jax.ShapeDtypeStruct((NROWP,), jnp.float32),
    ],
    scratch_shapes=[
        pltpu.VMEM((2, MVN), jnp.float32),
        pltpu.VMEM((2, MVN), jnp.float32),
        pltpu.SemaphoreType.DMA((2,)),
        pltpu.SemaphoreType.DMA((2,)),
    ],
)


def _sc_body(u_hbm, v_hbm, a_hbm, c_hbm, out_hbm,
             uidx_v, vidx_v, av, cv, out_v, sem_a, sem_c):
    wid = lax.axis_index("s") * NC + lax.axis_index("c")
    base = wid * BPW

    pltpu.sync_copy(u_hbm.at[pl.ds(base, BPW)], uidx_v)
    pltpu.sync_copy(v_hbm.at[pl.ds(base, BPW)], vidx_v)
    ca = pltpu.async_copy(a_hbm.at[uidx_v], av, sem_a)
    cc = pltpu.async_copy(c_hbm.at[vidx_v], cv, sem_c)
    ca.wait()
    cc.wait()

    def block(i, _):
        z = av[pl.ds(i * L, L)] + cv[pl.ds(i * L, L)]
        out_v[pl.ds(i * L, L)] = 1.0 / (1.0 + jnp.exp(-z))
        return _

    lax.fori_loop(0, NBLK, block, None)
    pltpu.sync_copy(out_v, out_hbm.at[pl.ds(base, BPW)])


@functools.partial(
    pl.kernel,
    out_type=jax.ShapeDtypeStruct((B,), jnp.float32),
    mesh=plsc.VectorSubcoreMesh(core_axis_name="c", subcore_axis_name="s"),
    compiler_params=pltpu.CompilerParams(
        use_tc_tiling_on_sc=False, needs_layout_passes=False),
    scratch_types=[
        pltpu.VMEM((BPW,), jnp.int32),
        pltpu.VMEM((BPW,), jnp.int32),
        pltpu.VMEM((BPW,), jnp.float32),
        pltpu.VMEM((BPW,), jnp.float32),
        pltpu.VMEM((BPW,), jnp.float32),
        pltpu.SemaphoreType.DMA,
        pltpu.SemaphoreType.DMA,
    ],
)
def _sc_lookup(*refs):
    _sc_body(*refs)


def kernel(x, W, H, lin_w, lin_b):
    u_idx = x[:, 0]
    v_idx = x[:, 1]
    wu = lin_w[:, :K].reshape(K, 1)
    wv = lin_w[:, K:].reshape(K, 1)
    bias = lin_b.reshape(1, 1)
    a, c = _mv_call(W.T, H.T, wu, wv, bias)
    return _sc_lookup(u_idx, v_idx, a, c)
